# baseline (speedup 1.0000x reference)
import functools

import jax
import jax.numpy as jnp
from jax import lax
from jax.experimental import pallas as pl
from jax.experimental.pallas import tpu as pltpu
from jax.experimental.pallas import tpu_sc as plsc

NUM_GRAPHS = 256
NC = 2
NS = 16
NW = NC * NS
L = 16

ACCW = 385
REDW = 384
RED = 3 * REDW

EB = 10000
EU = 5

NR = 3200


def _sc_body(pxf, txf, n2g, pq, tq, e2g,
             out,
             p_buf, t_buf, g_buf,
             np_buf, nt_buf, ng_buf,
             acc_x, acc_c, acc_q,
             red_buf, big_buf, res_buf,
             shared,
             sem0, sem1,
             *, e_per_w, n_last):
    c = lax.axis_index("c")
    s = lax.axis_index("s")
    wid = c * NS + s
    iota = lax.iota(jnp.int32, L)
    lane_base = iota * ACCW
    zeros = jnp.zeros((L,), jnp.float32)
    ones = jnp.ones((L,), jnp.float32)
    n_blk = e_per_w // EB
    sems = (sem0, sem1)

    eb_base = wid * e_per_w

    def start_block(b):
        slot = b % 2
        base = eb_base + b * EB
        return (
            pltpu.async_copy(pq.at[pl.ds(base, EB)],
                             p_buf.at[pl.ds(slot * EB, EB)], sems[slot]),
            pltpu.async_copy(tq.at[pl.ds(base, EB)],
                             t_buf.at[pl.ds(slot * EB, EB)], sems[slot]),
            pltpu.async_copy(e2g.at[pl.ds(base, EB)],
                             g_buf.at[pl.ds(slot * EB, EB)], sems[slot]),
        )

    inflight = {0: start_block(0)}

    def zbody(j, _):
        acc_x[pl.ds(j * L, L)] = zeros
        acc_c[pl.ds(j * L, L)] = zeros
        acc_q[pl.ds(j * L, L)] = zeros
        return 0
    lax.fori_loop(0, (L * ACCW) // L, zbody, 0)

    base3 = 3 * iota

    def node_part(nrows):
        rbase = wid * NR
        pltpu.sync_copy(pxf.at[pl.ds(3 * rbase, 3 * nrows)],
                        np_buf.at[pl.ds(0, 3 * nrows)])
        pltpu.sync_copy(txf.at[pl.ds(3 * rbase, 3 * nrows)],
                        nt_buf.at[pl.ds(0, 3 * nrows)])
        pltpu.sync_copy(n2g.at[pl.ds(rbase, nrows)],
                        ng_buf.at[pl.ds(0, nrows)])

        def nbody(v, _):
            eb = v * (3 * L) + base3
            p0 = plsc.load_gather(np_buf, [eb])
            t0 = plsc.load_gather(nt_buf, [eb])
            p1 = plsc.load_gather(np_buf, [eb + 1])
            t1 = plsc.load_gather(nt_buf, [eb + 1])
            p2 = plsc.load_gather(np_buf, [eb + 2])
            t2 = plsc.load_gather(nt_buf, [eb + 2])
            d0 = p0 - t0
            d1 = p1 - t1
            d2 = p2 - t2
            e2 = d0 * d0 + d1 * d1 + d2 * d2
            idx = lane_base + ng_buf[pl.ds(v * L, L)]
            plsc.addupdate_scatter(acc_x, [idx], e2)
            plsc.addupdate_scatter(acc_c, [idx], ones)
            return 0
        lax.fori_loop(0, nrows // L, nbody, 0)

    @pl.when(wid < NW - 1)
    def _():
        node_part(NR)

    @pl.when(wid == NW - 1)
    def _():
        node_part(n_last)

    for b in range(n_blk):
        slot = b % 2
        if b + 1 < n_blk:
            inflight[b + 1] = start_block(b + 1)
        for h in inflight.pop(b):
            h.wait()
        sbase = slot * EB

        def ebody(v, _):
            off = sbase + v * (EU * L)
            for u in range(EU):
                o = off + u * L
                p = p_buf[pl.ds(o, L)]
                t = t_buf[pl.ds(o, L)]
                g = g_buf[pl.ds(o, L)]
                d = p - t
                plsc.addupdate_scatter(acc_q, [lane_base + g], d * d)
            return 0
        lax.fori_loop(0, EB // (EU * L), ebody, 0)

    def rbody(j, _):
        col = j * L
        sx = zeros
        sc = zeros
        sq = zeros
        for i in range(L):
            gidx = i * ACCW + col + iota
            sx = sx + plsc.load_gather(acc_x, [gidx])
            sc = sc + plsc.load_gather(acc_c, [gidx])
            sq = sq + plsc.load_gather(acc_q, [gidx])
        red_buf[pl.ds(col, L)] = sx
        red_buf[pl.ds(REDW + col, L)] = sc
        red_buf[pl.ds(2 * REDW + col, L)] = sq
        return 0
    lax.fori_loop(0, REDW // L, rbody, 0)

    pltpu.sync_copy(red_buf, shared.at[pl.ds(s * RED, RED)])
    plsc.subcore_barrier()

    @pl.when(s == 0)
    def _():
        pltpu.sync_copy(shared, big_buf)

        def fbody(j, _):
            col = j * L
            acc = zeros
            for i in range(NS):
                acc = acc + big_buf[pl.ds(i * RED + col, L)]
            res_buf[pl.ds(col, L)] = acc
            return 0
        lax.fori_loop(0, RED // L, fbody, 0)
        pltpu.sync_copy(res_buf, out.at[pl.ds(c * RED, RED)])


def _fin_body(p_ref, o1_ref, o2_ref):
    p = p_ref[...]
    t = p[0] + p[1]
    sx = t[0:1, :]
    cnt = t[1:2, :]
    sq = t[2:3, :]
    rmsd = jnp.sqrt(sx / jnp.maximum(cnt, 1.0))
    o1_ref[0, 0] = jnp.sum(rmsd) * (1.0 / NUM_GRAPHS)
    o2_ref[0, 0] = jnp.sum(jnp.sqrt(sq)) * (1.0 / NUM_GRAPHS)


def kernel(pred_x, pred_q, target_x, target_q, edge2graph, node2graph,
           atom_type, edge_r, edge_p):
    n = node2graph.shape[0]
    e = edge2graph.shape[0]
    assert e % (NW * L) == 0 and (e // NW) % EB == 0
    e_per_w = e // NW

    n_last = n - (NW - 1) * NR
    assert 0 < n_last <= NR and n_last % L == 0

    pxf = pred_x.reshape(3 * n)
    txf = target_x.reshape(3 * n)

    sc_call = pl.kernel(
        functools.partial(_sc_body, e_per_w=e_per_w, n_last=n_last),
        out_type=jax.ShapeDtypeStruct((NC * RED,), jnp.float32),
        mesh=plsc.VectorSubcoreMesh(core_axis_name="c", subcore_axis_name="s"),
        compiler_params=pltpu.CompilerParams(needs_layout_passes=False,
                                             use_tc_tiling_on_sc=False),
        scratch_types=[
            pltpu.VMEM((2 * EB,), jnp.float32),
            pltpu.VMEM((2 * EB,), jnp.float32),
            pltpu.VMEM((2 * EB,), jnp.int32),
            pltpu.VMEM((3 * NR,), jnp.float32),
            pltpu.VMEM((3 * NR,), jnp.float32),
            pltpu.VMEM((NR,), jnp.int32),
            pltpu.VMEM((L * ACCW,), jnp.float32),
            pltpu.VMEM((L * ACCW,), jnp.float32),
            pltpu.VMEM((L * ACCW,), jnp.float32),
            pltpu.VMEM((RED,), jnp.float32),
            pltpu.VMEM((NS * RED,), jnp.float32),
            pltpu.VMEM((RED,), jnp.float32),
            pltpu.VMEM_SHARED((NS * RED,), jnp.float32),
            pltpu.SemaphoreType.DMA,
            pltpu.SemaphoreType.DMA,
        ],
    )
    partials = sc_call(pxf, txf, node2graph, pred_q, target_q, edge2graph)
    partials = partials.reshape(NC, 3, REDW)

    r1, r2 = pl.pallas_call(
        _fin_body,
        out_shape=(jax.ShapeDtypeStruct((1, 1), jnp.float32),
                   jax.ShapeDtypeStruct((1, 1), jnp.float32)),
        in_specs=[pl.BlockSpec(memory_space=pltpu.VMEM)],
        out_specs=(pl.BlockSpec(memory_space=pltpu.SMEM),
                   pl.BlockSpec(memory_space=pltpu.SMEM)),
    )(partials)
    return (r1[0, 0], r2[0, 0])


# speedup vs baseline: 2.4427x; 2.4427x over previous
import functools

import jax
import jax.numpy as jnp
from jax import lax
from jax.experimental import pallas as pl
from jax.experimental.pallas import tpu as pltpu
from jax.experimental.pallas import tpu_sc as plsc

NUM_GRAPHS = 256
NC = 2
NS = 16
NW = NC * NS
L = 16

ACCW = 385
REDW = 384
RED = 3 * REDW

EB = 10000
EU = 5

NR = 3200


def _sc_body(pxf, txf, n2g, pq, tq, e2g,
             out,
             p_buf, t_buf, g_buf,
             np_buf, nt_buf, ng_buf,
             acc_x, acc_c, acc_q,
             red_buf, big_buf, res_buf,
             shared,
             sem0, sem1,
             *, e_per_w, n_last, n_total):
    c = lax.axis_index("c")
    s = lax.axis_index("s")
    wid = c * NS + s
    iota = lax.iota(jnp.int32, L)
    lane_base = iota * ACCW
    zeros = jnp.zeros((L,), jnp.float32)
    ones = jnp.ones((L,), jnp.float32)
    n_blk = e_per_w // EB
    sems = (sem0, sem1)

    eb_base = wid * e_per_w

    def start_block(b):
        slot = b % 2
        base = eb_base + b * EB
        return (
            pltpu.async_copy(pq.at[pl.ds(base, EB)],
                             p_buf.at[pl.ds(slot * EB, EB)], sems[slot]),
            pltpu.async_copy(tq.at[pl.ds(base, EB)],
                             t_buf.at[pl.ds(slot * EB, EB)], sems[slot]),
            pltpu.async_copy(e2g.at[pl.ds(base, EB)],
                             g_buf.at[pl.ds(slot * EB, EB)], sems[slot]),
        )

    inflight = {0: start_block(0)}

    def zbody(j, _):
        acc_x[pl.ds(j * L, L)] = zeros
        acc_c[pl.ds(j * L, L)] = zeros
        acc_q[pl.ds(j * L, L)] = zeros
        return 0
    lax.fori_loop(0, (L * ACCW) // L, zbody, 0)

    def node_part(nrows):
        rbase = wid * NR
        for plane in range(3):
            pltpu.sync_copy(pxf.at[pl.ds(plane * n_total + rbase, nrows)],
                            np_buf.at[pl.ds(plane * NR, nrows)])
            pltpu.sync_copy(txf.at[pl.ds(plane * n_total + rbase, nrows)],
                            nt_buf.at[pl.ds(plane * NR, nrows)])
        pltpu.sync_copy(n2g.at[pl.ds(rbase, nrows)],
                        ng_buf.at[pl.ds(0, nrows)])

        def nbody(v, _):
            o = v * L
            d0 = np_buf[pl.ds(o, L)] - nt_buf[pl.ds(o, L)]
            d1 = np_buf[pl.ds(NR + o, L)] - nt_buf[pl.ds(NR + o, L)]
            d2 = np_buf[pl.ds(2 * NR + o, L)] - nt_buf[pl.ds(2 * NR + o, L)]
            e2 = d0 * d0 + d1 * d1 + d2 * d2
            idx = lane_base + ng_buf[pl.ds(o, L)]
            plsc.addupdate_scatter(acc_x, [idx], e2)
            plsc.addupdate_scatter(acc_c, [idx], ones)
            return 0
        lax.fori_loop(0, nrows // L, nbody, 0)

    @pl.when(wid < NW - 1)
    def _():
        node_part(NR)

    @pl.when(wid == NW - 1)
    def _():
        node_part(n_last)

    for b in range(n_blk):
        slot = b % 2
        if b + 1 < n_blk:
            inflight[b + 1] = start_block(b + 1)
        for h in inflight.pop(b):
            h.wait()
        sbase = slot * EB

        def ebody(v, _):
            off = sbase + v * (EU * L)
            for u in range(EU):
                o = off + u * L
                p = p_buf[pl.ds(o, L)]
                t = t_buf[pl.ds(o, L)]
                g = g_buf[pl.ds(o, L)]
                d = p - t
                plsc.addupdate_scatter(acc_q, [lane_base + g], d * d)
            return 0
        lax.fori_loop(0, EB // (EU * L), ebody, 0)

    def rbody(j, _):
        col = j * L
        sx = zeros
        sc = zeros
        sq = zeros
        for i in range(L):
            gidx = i * ACCW + col + iota
            sx = sx + plsc.load_gather(acc_x, [gidx])
            sc = sc + plsc.load_gather(acc_c, [gidx])
            sq = sq + plsc.load_gather(acc_q, [gidx])
        red_buf[pl.ds(col, L)] = sx
        red_buf[pl.ds(REDW + col, L)] = sc
        red_buf[pl.ds(2 * REDW + col, L)] = sq
        return 0
    lax.fori_loop(0, REDW // L, rbody, 0)

    pltpu.sync_copy(red_buf, shared.at[pl.ds(s * RED, RED)])
    plsc.subcore_barrier()

    @pl.when(s == 0)
    def _():
        pltpu.sync_copy(shared, big_buf)

        def fbody(j, _):
            col = j * L
            acc = zeros
            for i in range(NS):
                acc = acc + big_buf[pl.ds(i * RED + col, L)]
            res_buf[pl.ds(col, L)] = acc
            return 0
        lax.fori_loop(0, RED // L, fbody, 0)
        pltpu.sync_copy(res_buf, out.at[pl.ds(c * RED, RED)])


def _fin_body(p_ref, o1_ref, o2_ref):
    p = p_ref[...]
    t = p[0] + p[1]
    sx = t[0:1, :]
    cnt = t[1:2, :]
    sq = t[2:3, :]
    rmsd = jnp.sqrt(sx / jnp.maximum(cnt, 1.0))
    o1_ref[0, 0] = jnp.sum(rmsd) * (1.0 / NUM_GRAPHS)
    o2_ref[0, 0] = jnp.sum(jnp.sqrt(sq)) * (1.0 / NUM_GRAPHS)


def kernel(pred_x, pred_q, target_x, target_q, edge2graph, node2graph,
           atom_type, edge_r, edge_p):
    n = node2graph.shape[0]
    e = edge2graph.shape[0]
    assert e % (NW * L) == 0 and (e // NW) % EB == 0
    e_per_w = e // NW

    n_last = n - (NW - 1) * NR
    assert 0 < n_last <= NR and n_last % L == 0

    pxf = pred_x.T.reshape(3 * n)
    txf = target_x.T.reshape(3 * n)

    sc_call = pl.kernel(
        functools.partial(_sc_body, e_per_w=e_per_w, n_last=n_last,
                          n_total=n),
        out_type=jax.ShapeDtypeStruct((NC * RED,), jnp.float32),
        mesh=plsc.VectorSubcoreMesh(core_axis_name="c", subcore_axis_name="s"),
        compiler_params=pltpu.CompilerParams(needs_layout_passes=False,
                                             use_tc_tiling_on_sc=False),
        scratch_types=[
            pltpu.VMEM((2 * EB,), jnp.float32),
            pltpu.VMEM((2 * EB,), jnp.float32),
            pltpu.VMEM((2 * EB,), jnp.int32),
            pltpu.VMEM((3 * NR,), jnp.float32),
            pltpu.VMEM((3 * NR,), jnp.float32),
            pltpu.VMEM((NR,), jnp.int32),
            pltpu.VMEM((L * ACCW,), jnp.float32),
            pltpu.VMEM((L * ACCW,), jnp.float32),
            pltpu.VMEM((L * ACCW,), jnp.float32),
            pltpu.VMEM((RED,), jnp.float32),
            pltpu.VMEM((NS * RED,), jnp.float32),
            pltpu.VMEM((RED,), jnp.float32),
            pltpu.VMEM_SHARED((NS * RED,), jnp.float32),
            pltpu.SemaphoreType.DMA,
            pltpu.SemaphoreType.DMA,
        ],
    )
    partials = sc_call(pxf, txf, node2graph, pred_q, target_q, edge2graph)
    partials = partials.reshape(NC, 3, REDW)

    r1, r2 = pl.pallas_call(
        _fin_body,
        out_shape=(jax.ShapeDtypeStruct((1, 1), jnp.float32),
                   jax.ShapeDtypeStruct((1, 1), jnp.float32)),
        in_specs=[pl.BlockSpec(memory_space=pltpu.VMEM)],
        out_specs=(pl.BlockSpec(memory_space=pltpu.SMEM),
                   pl.BlockSpec(memory_space=pltpu.SMEM)),
    )(partials)
    return (r1[0, 0], r2[0, 0])


# speedup vs baseline: 3.3729x; 1.3808x over previous
import functools

import jax
import jax.numpy as jnp
from jax import lax
from jax.experimental import pallas as pl
from jax.experimental.pallas import tpu as pltpu
from jax.experimental.pallas import tpu_sc as plsc

NUM_GRAPHS = 256
NC = 2
NS = 16
NW = NC * NS
L = 16

ACCW = 385
REDW = 384
RED = 3 * REDW

EB = 10000
EU = 5

NR = 3200


def _sc_body(pxf, txf, n2g, pq, tq, e2g,
             out,
             p_buf, t_buf, g_buf,
             np_buf, nt_buf, ng_buf,
             acc_x, acc_c, acc_q,
             red_buf, big_buf, res_buf,
             shared,
             sem0, sem1,
             *, e_per_w, n_last, n_total):
    c = lax.axis_index("c")
    s = lax.axis_index("s")
    wid = c * NS + s
    iota = lax.iota(jnp.int32, L)
    lane_base = iota * ACCW
    zeros = jnp.zeros((L,), jnp.float32)
    ones = jnp.ones((L,), jnp.float32)
    n_blk = e_per_w // EB
    sems = (sem0, sem1)

    eb_base = wid * e_per_w

    def start_block(b):
        slot = b % 2
        base = eb_base + b * EB
        return (
            pltpu.async_copy(pq.at[pl.ds(base, EB)],
                             p_buf.at[pl.ds(slot * EB, EB)], sems[slot]),
            pltpu.async_copy(tq.at[pl.ds(base, EB)],
                             t_buf.at[pl.ds(slot * EB, EB)], sems[slot]),
            pltpu.async_copy(e2g.at[pl.ds(base, EB)],
                             g_buf.at[pl.ds(slot * EB, EB)], sems[slot]),
        )

    inflight = {0: start_block(0)}

    def zbody(j, _):
        acc_x[pl.ds(j * L, L)] = zeros
        acc_c[pl.ds(j * L, L)] = zeros
        acc_q[pl.ds(j * L, L)] = zeros
        return 0
    lax.fori_loop(0, (L * ACCW) // L, zbody, 0)

    def node_part(nrows):
        rbase = wid * NR
        for plane in range(3):
            pltpu.sync_copy(pxf.at[pl.ds(plane * n_total + rbase, nrows)],
                            np_buf.at[pl.ds(plane * NR, nrows)])
            pltpu.sync_copy(txf.at[pl.ds(plane * n_total + rbase, nrows)],
                            nt_buf.at[pl.ds(plane * NR, nrows)])
        pltpu.sync_copy(n2g.at[pl.ds(rbase, nrows)],
                        ng_buf.at[pl.ds(0, nrows)])

        def nbody(v, _):
            o = v * L
            d0 = np_buf[pl.ds(o, L)] - nt_buf[pl.ds(o, L)]
            d1 = np_buf[pl.ds(NR + o, L)] - nt_buf[pl.ds(NR + o, L)]
            d2 = np_buf[pl.ds(2 * NR + o, L)] - nt_buf[pl.ds(2 * NR + o, L)]
            e2 = d0 * d0 + d1 * d1 + d2 * d2
            idx = lane_base + ng_buf[pl.ds(o, L)]
            plsc.addupdate_scatter(acc_x, [idx], e2)
            plsc.addupdate_scatter(acc_c, [idx], ones)
            return 0
        lax.fori_loop(0, nrows // L, nbody, 0)

    @pl.when(wid < NW - 1)
    def _():
        node_part(NR)

    @pl.when(wid == NW - 1)
    def _():
        node_part(n_last)

    for b in range(n_blk):
        slot = b % 2
        if b + 1 < n_blk:
            inflight[b + 1] = start_block(b + 1)
        for h in inflight.pop(b):
            h.wait()
        sbase = slot * EB

        def ebody(v, _):
            off = sbase + v * (EU * L)
            ps = [p_buf[pl.ds(off + u * L, L)] for u in range(EU)]
            ts = [t_buf[pl.ds(off + u * L, L)] for u in range(EU)]
            gs = [g_buf[pl.ds(off + u * L, L)] for u in range(EU)]
            idxs = [lane_base + g for g in gs]
            sqs = [(p - t) * (p - t) for p, t in zip(ps, ts)]
            for u in range(EU):
                plsc.addupdate_scatter(acc_q, [idxs[u]], sqs[u])
            return 0
        lax.fori_loop(0, EB // (EU * L), ebody, 0)

    def rbody(j, _):
        col = j * L
        sx = zeros
        sc = zeros
        sq = zeros
        for i in range(L):
            gidx = i * ACCW + col + iota
            sx = sx + plsc.load_gather(acc_x, [gidx])
            sc = sc + plsc.load_gather(acc_c, [gidx])
            sq = sq + plsc.load_gather(acc_q, [gidx])
        red_buf[pl.ds(col, L)] = sx
        red_buf[pl.ds(REDW + col, L)] = sc
        red_buf[pl.ds(2 * REDW + col, L)] = sq
        return 0
    lax.fori_loop(0, REDW // L, rbody, 0)

    pltpu.sync_copy(red_buf, shared.at[pl.ds(s * RED, RED)])
    plsc.subcore_barrier()

    @pl.when(s == 0)
    def _():
        pltpu.sync_copy(shared, big_buf)

        def fbody(j, _):
            col = j * L
            acc = zeros
            for i in range(NS):
                acc = acc + big_buf[pl.ds(i * RED + col, L)]
            res_buf[pl.ds(col, L)] = acc
            return 0
        lax.fori_loop(0, RED // L, fbody, 0)
        pltpu.sync_copy(res_buf, out.at[pl.ds(c * RED, RED)])


def _fin_body(p_ref, o1_ref, o2_ref):
    p = p_ref[...]
    t = p[0] + p[1]
    sx = t[0:1, :]
    cnt = t[1:2, :]
    sq = t[2:3, :]
    rmsd = jnp.sqrt(sx / jnp.maximum(cnt, 1.0))
    o1_ref[0, 0] = jnp.sum(rmsd) * (1.0 / NUM_GRAPHS)
    o2_ref[0, 0] = jnp.sum(jnp.sqrt(sq)) * (1.0 / NUM_GRAPHS)


def kernel(pred_x, pred_q, target_x, target_q, edge2graph, node2graph,
           atom_type, edge_r, edge_p):
    n = node2graph.shape[0]
    e = edge2graph.shape[0]
    assert e % (NW * L) == 0 and (e // NW) % EB == 0
    e_per_w = e // NW

    n_last = n - (NW - 1) * NR
    assert 0 < n_last <= NR and n_last % L == 0

    pxf = pred_x.T.reshape(3 * n)
    txf = target_x.T.reshape(3 * n)

    sc_call = pl.kernel(
        functools.partial(_sc_body, e_per_w=e_per_w, n_last=n_last,
                          n_total=n),
        out_type=jax.ShapeDtypeStruct((NC * RED,), jnp.float32),
        mesh=plsc.VectorSubcoreMesh(core_axis_name="c", subcore_axis_name="s"),
        compiler_params=pltpu.CompilerParams(needs_layout_passes=False,
                                             use_tc_tiling_on_sc=False),
        scratch_types=[
            pltpu.VMEM((2 * EB,), jnp.float32),
            pltpu.VMEM((2 * EB,), jnp.float32),
            pltpu.VMEM((2 * EB,), jnp.int32),
            pltpu.VMEM((3 * NR,), jnp.float32),
            pltpu.VMEM((3 * NR,), jnp.float32),
            pltpu.VMEM((NR,), jnp.int32),
            pltpu.VMEM((L * ACCW,), jnp.float32),
            pltpu.VMEM((L * ACCW,), jnp.float32),
            pltpu.VMEM((L * ACCW,), jnp.float32),
            pltpu.VMEM((RED,), jnp.float32),
            pltpu.VMEM((NS * RED,), jnp.float32),
            pltpu.VMEM((RED,), jnp.float32),
            pltpu.VMEM_SHARED((NS * RED,), jnp.float32),
            pltpu.SemaphoreType.DMA,
            pltpu.SemaphoreType.DMA,
        ],
    )
    partials = sc_call(pxf, txf, node2graph, pred_q, target_q, edge2graph)
    partials = partials.reshape(NC, 3, REDW)

    r1, r2 = pl.pallas_call(
        _fin_body,
        out_shape=(jax.ShapeDtypeStruct((1, 1), jnp.float32),
                   jax.ShapeDtypeStruct((1, 1), jnp.float32)),
        in_specs=[pl.BlockSpec(memory_space=pltpu.VMEM)],
        out_specs=(pl.BlockSpec(memory_space=pltpu.SMEM),
                   pl.BlockSpec(memory_space=pltpu.SMEM)),
    )(partials)
    return (r1[0, 0], r2[0, 0])
